# baseline (speedup 1.0000x reference)
import functools

import jax
import jax.numpy as jnp
from jax import lax
from jax.experimental import pallas as pl
from jax.experimental.pallas import tpu as pltpu
from jax.experimental.pallas import tpu_sc as plsc

F32 = jnp.float32
I32 = jnp.int32

NC = 2
NS = 16
NW = NC * NS
LANES = 16
CH = 128
RB = 1024

_MESH = dict(core_axis_name="c", subcore_axis_name="s")


def _pad_to(v, m):
    return ((v + m - 1) // m) * m


def _zero_fill(ref):
    rows, cols = ref.shape
    z = jnp.zeros((LANES,), F32)

    def body(i, _):
        r = i // (cols // LANES)
        c = lax.rem(i, cols // LANES)
        ref[r, pl.ds(c * LANES, LANES)] = z
        return 0

    lax.fori_loop(0, rows * (cols // LANES), body, 0)


def _sc_degree(e3, n_pad):
    _, nw, nch, ch = e3.shape
    rpt = n_pad // NS

    def body(e_hbm, out_hbm, dst_v, ones_v, acc_sh):
        c = lax.axis_index("c")
        s = lax.axis_index("s")
        w = s * NC + c
        pltpu.sync_copy(e_hbm.at[1, w], dst_v)
        _zero_fill(ones_v)
        for k in range(rpt // CH):
            pltpu.sync_copy(ones_v, acc_sh.at[pl.ds(s * rpt + k * CH, CH)])
        one = jnp.ones((LANES,), F32)

        def fill(i, _):
            ones_v[i, pl.ds(0, LANES)] = one
            return 0

        lax.fori_loop(0, CH, fill, 0)
        plsc.subcore_barrier()

        def step(j, _):
            pltpu.sync_copy(ones_v, acc_sh.at[dst_v.at[j]], add=True)
            return 0

        lax.fori_loop(0, nch, step, 0)
        plsc.subcore_barrier()
        for k in range(rpt // CH):
            pltpu.sync_copy(acc_sh.at[pl.ds(s * rpt + k * CH, CH)],
                            out_hbm.at[c, pl.ds(s * rpt + k * CH, CH)])

    return pl.kernel(
        body,
        out_type=jax.ShapeDtypeStruct((NC, n_pad, LANES), F32),
        mesh=plsc.VectorSubcoreMesh(**_MESH),
        compiler_params=pltpu.CompilerParams(use_tc_tiling_on_sc=False),
        scratch_types=[
            pltpu.VMEM((nch, ch), I32),
            pltpu.VMEM((CH, LANES), F32),
            pltpu.VMEM_SHARED((n_pad, LANES), F32),
        ],
    )(e3)


def _sc_scatter(table, e3):
    n_pad, h = table.shape
    _, nw, nch, ch = e3.shape
    rpt = n_pad // NS

    def body(tab_hbm, e_hbm, out_hbm, src_v, dst_v, rows_v, acc_sh,
             tab_sh, sem, gsem, ssem):
        c = lax.axis_index("c")
        s = lax.axis_index("s")
        w = s * NC + c
        pltpu.async_copy(e_hbm.at[0, w], src_v, sem)
        pltpu.async_copy(e_hbm.at[1, w], dst_v, sem)
        pltpu.async_copy(tab_hbm.at[pl.ds(s * rpt, rpt)],
                         tab_sh.at[pl.ds(s * rpt, rpt)], sem)
        _zero_fill(rows_v.at[0])
        for k in range(rpt // CH):
            pltpu.sync_copy(rows_v.at[0], acc_sh.at[pl.ds(s * rpt + k * CH, CH)])
        pltpu.make_async_copy(e_hbm.at[0, w], src_v, sem).wait()
        pltpu.make_async_copy(e_hbm.at[1, w], dst_v, sem).wait()
        pltpu.make_async_copy(tab_hbm.at[pl.ds(s * rpt, rpt)],
                              tab_sh.at[pl.ds(s * rpt, rpt)], sem).wait()
        plsc.subcore_barrier()

        for p in range(3):
            pltpu.async_copy(tab_sh.at[src_v.at[p]], rows_v.at[p], gsem.at[p])

        def step(j, _):
            b = lax.rem(j, 3)
            pltpu.make_async_copy(tab_sh.at[src_v.at[j]], rows_v.at[b],
                                  gsem.at[b]).wait()
            pltpu.async_copy(rows_v.at[b], acc_sh.at[dst_v.at[j]], ssem.at[b],
                             add=True)

            @pl.when(j + 3 < nch)
            def _():
                pltpu.make_async_copy(rows_v.at[b], acc_sh.at[dst_v.at[j]],
                                      ssem.at[b]).wait()
                pltpu.async_copy(tab_sh.at[src_v.at[j + 3]], rows_v.at[b],
                                 gsem.at[b])

            return 0

        lax.fori_loop(0, nch, step, 0)
        for p in range(3):
            j = nch - 3 + p
            b = j % 3
            pltpu.make_async_copy(rows_v.at[b], acc_sh.at[dst_v.at[j]],
                                  ssem.at[b]).wait()
        plsc.subcore_barrier()
        for k in range(rpt // CH):
            pltpu.async_copy(acc_sh.at[pl.ds(s * rpt + k * CH, CH)],
                             out_hbm.at[c, pl.ds(s * rpt + k * CH, CH)], sem)
        for k in range(rpt // CH):
            pltpu.make_async_copy(acc_sh.at[pl.ds(s * rpt + k * CH, CH)],
                                  out_hbm.at[c, pl.ds(s * rpt + k * CH, CH)],
                                  sem).wait()

    return pl.kernel(
        body,
        out_type=jax.ShapeDtypeStruct((NC, n_pad, h), F32),
        mesh=plsc.VectorSubcoreMesh(**_MESH),
        compiler_params=pltpu.CompilerParams(use_tc_tiling_on_sc=False),
        scratch_types=[
            pltpu.VMEM((nch, ch), I32),
            pltpu.VMEM((nch, ch), I32),
            pltpu.VMEM((3, CH, h), F32),
            pltpu.VMEM_SHARED((n_pad, h), F32),
            pltpu.VMEM_SHARED((n_pad, h), F32),
            pltpu.SemaphoreType.DMA,
            pltpu.SemaphoreType.DMA((3,)),
            pltpu.SemaphoreType.DMA((3,)),
        ],
    )(table, e3)


def _tc_k1(x, w1t, degp, n_pad):
    d = x.shape[1]
    h = w1t.shape[1]
    grid = (n_pad // RB,)

    def body(x_ref, w_ref, dg_ref, hp1s_ref, dinv_ref):
        dg = dg_ref[...]
        deg = dg[0, :, 0:1] + dg[1, :, 0:1] + 1.0
        dinv = lax.rsqrt(deg)
        hp1 = jnp.dot(x_ref[...], w_ref[...], preferred_element_type=F32)
        hp1s_ref[...] = hp1 * dinv
        dinv_ref[...] = dinv

    return pl.pallas_call(
        body,
        grid=grid,
        in_specs=[
            pl.BlockSpec((RB, d), lambda i: (i, 0)),
            pl.BlockSpec((d, h), lambda i: (0, 0)),
            pl.BlockSpec((NC, RB, LANES), lambda i: (0, i, 0)),
        ],
        out_specs=[
            pl.BlockSpec((RB, h), lambda i: (i, 0)),
            pl.BlockSpec((RB, 1), lambda i: (i, 0)),
        ],
        out_shape=[
            jax.ShapeDtypeStruct((n_pad, h), F32),
            jax.ShapeDtypeStruct((n_pad, 1), F32),
        ],
    )(x, w1t, degp)


def _tc_k2(s1, hp1s, dinv, b1, g1, be1, rm1, rv1, w2t):
    n_pad, h = hp1s.shape
    grid = (n_pad // RB,)

    def body(s_ref, hp_ref, dv_ref, b_ref, g_ref, be_ref, rm_ref, rv_ref,
             w_ref, out_ref):
        s = s_ref[...]
        dinv = dv_ref[...]
        conv = dinv * (s[0] + s[1] + hp_ref[...]) + b_ref[...]
        scale = g_ref[...] * lax.rsqrt(rv_ref[...] + 1e-5)
        hh = jnp.maximum((conv - rm_ref[...]) * scale + be_ref[...], 0.0)
        hp2 = jnp.dot(hh, w_ref[...], preferred_element_type=F32)
        out_ref[...] = hp2 * dinv

    vec = lambda: pl.BlockSpec((1, h), lambda i: (0, 0))
    return pl.pallas_call(
        body,
        grid=grid,
        in_specs=[
            pl.BlockSpec((NC, RB, h), lambda i: (0, i, 0)),
            pl.BlockSpec((RB, h), lambda i: (i, 0)),
            pl.BlockSpec((RB, 1), lambda i: (i, 0)),
            vec(), vec(), vec(), vec(), vec(),
            pl.BlockSpec((h, h), lambda i: (0, 0)),
        ],
        out_specs=pl.BlockSpec((RB, h), lambda i: (i, 0)),
        out_shape=jax.ShapeDtypeStruct((n_pad, h), F32),
    )(s1, hp1s, dinv, b1, g1, be1, rm1, rv1, w2t)


def _tc_k3(s2, hp2s, dinv, b2, g2, be2, rm2, rv2, fc1t, fc1b, fc2t, fc2b, n):
    n_pad, h = hp2s.shape
    h2 = fc1t.shape[1]
    grid = (n_pad // RB,)

    def body(s_ref, hp_ref, dv_ref, b_ref, g_ref, be_ref, rm_ref, rv_ref,
             f1_ref, f1b_ref, f2_ref, f2b_ref, out_ref):
        s = s_ref[...]
        dinv = dv_ref[...]
        conv = dinv * (s[0] + s[1] + hp_ref[...]) + b_ref[...]
        scale = g_ref[...] * lax.rsqrt(rv_ref[...] + 1e-5)
        hh = jnp.maximum((conv - rm_ref[...]) * scale + be_ref[...], 0.0)
        hh = jnp.maximum(
            jnp.dot(hh, f1_ref[...], preferred_element_type=F32) + f1b_ref[...],
            0.0)
        out_ref[...] = (jnp.dot(hh, f2_ref[...], preferred_element_type=F32)
                        + f2b_ref[...])

    vec = lambda k: pl.BlockSpec((1, k), lambda i: (0, 0))
    return pl.pallas_call(
        body,
        grid=grid,
        in_specs=[
            pl.BlockSpec((NC, RB, h), lambda i: (0, i, 0)),
            pl.BlockSpec((RB, h), lambda i: (i, 0)),
            pl.BlockSpec((RB, 1), lambda i: (i, 0)),
            vec(h), vec(h), vec(h), vec(h), vec(h),
            pl.BlockSpec((h, h2), lambda i: (0, 0)),
            vec(h2),
            pl.BlockSpec((h2, 1), lambda i: (0, 0)),
            vec(1),
        ],
        out_specs=pl.BlockSpec((RB, 1), lambda i: (i, 0)),
        out_shape=jax.ShapeDtypeStruct((n, 1), F32),
    )(s2, hp2s, dinv, b2, g2, be2, rm2, rv2, fc1t, fc1b, fc2t, fc2b)


def kernel(x, edge_index, W1, b1, W2, b2, g1, be1, rm1, rv1, g2, be2, rm2,
           rv2, fc1_w, fc1_b, fc2_w, fc2_b):
    n, d = x.shape
    e = edge_index.shape[1]

    n_pad = _pad_to(n + 1, NS * CH)
    e_pad = _pad_to(e, NW * CH)
    nch = e_pad // (NW * CH)

    e3 = jnp.pad(edge_index.astype(I32), ((0, 0), (0, e_pad - e)),
                 constant_values=n).reshape(2, NW, nch, CH)

    row = lambda v: v.reshape(1, -1)

    degp = _sc_degree(e3, n_pad)
    hp1s, dinv = _tc_k1(x, W1.T, degp, n_pad)
    s1 = _sc_scatter(hp1s, e3)
    hp2s = _tc_k2(s1, hp1s, dinv, row(b1), row(g1), row(be1), row(rm1),
                  row(rv1), W2.T)
    s2 = _sc_scatter(hp2s, e3)
    return _tc_k3(s2, hp2s, dinv, row(b2), row(g2), row(be2), row(rm2),
                  row(rv2), fc1_w.T, row(fc1_b), fc2_w.T, row(fc2_b), n)


# speedup vs baseline: 1.0308x; 1.0308x over previous
import functools

import jax
import jax.numpy as jnp
from jax import lax
from jax.experimental import pallas as pl
from jax.experimental.pallas import tpu as pltpu
from jax.experimental.pallas import tpu_sc as plsc

F32 = jnp.float32
I32 = jnp.int32

NC = 2
NS = 16
NW = NC * NS
LANES = 16
CH = 128
RB = 2560

_MESH = dict(core_axis_name="c", subcore_axis_name="s")


def _pad_to(v, m):
    return ((v + m - 1) // m) * m


def _zero_fill(ref):
    rows, cols = ref.shape
    z = jnp.zeros((LANES,), F32)

    def body(i, _):
        r = i // (cols // LANES)
        c = lax.rem(i, cols // LANES)
        ref[r, pl.ds(c * LANES, LANES)] = z
        return 0

    lax.fori_loop(0, rows * (cols // LANES), body, 0)


def _sc_degree(e3, n_pad):
    _, nw, nch, ch = e3.shape
    rpt = n_pad // NS

    def body(e_hbm, out_hbm, dst_v, ones_v, acc_sh):
        c = lax.axis_index("c")
        s = lax.axis_index("s")
        w = s * NC + c
        pltpu.sync_copy(e_hbm.at[1, w], dst_v)
        _zero_fill(ones_v)
        for k in range(rpt // CH):
            pltpu.sync_copy(ones_v, acc_sh.at[pl.ds(s * rpt + k * CH, CH)])
        one = jnp.ones((LANES,), F32)

        def fill(i, _):
            ones_v[i, pl.ds(0, LANES)] = one
            return 0

        lax.fori_loop(0, CH, fill, 0)
        plsc.subcore_barrier()

        def step(j, _):
            pltpu.sync_copy(ones_v, acc_sh.at[dst_v.at[j]], add=True)
            return 0

        lax.fori_loop(0, nch, step, 0)
        plsc.subcore_barrier()
        for k in range(rpt // CH):
            pltpu.sync_copy(acc_sh.at[pl.ds(s * rpt + k * CH, CH)],
                            out_hbm.at[c, pl.ds(s * rpt + k * CH, CH)])

    return pl.kernel(
        body,
        out_type=jax.ShapeDtypeStruct((NC, n_pad, LANES), F32),
        mesh=plsc.VectorSubcoreMesh(**_MESH),
        compiler_params=pltpu.CompilerParams(use_tc_tiling_on_sc=False),
        scratch_types=[
            pltpu.VMEM((nch, ch), I32),
            pltpu.VMEM((CH, LANES), F32),
            pltpu.VMEM_SHARED((n_pad, LANES), F32),
        ],
    )(e3)


def _sc_scatter(table, e3):
    n_pad, h = table.shape
    _, nw, nch, ch = e3.shape
    rpt = n_pad // NS

    def body(tab_hbm, e_hbm, out_hbm, src_v, dst_v, rows_v, acc_sh,
             tab_sh, sem, gsem, ssem):
        c = lax.axis_index("c")
        s = lax.axis_index("s")
        w = s * NC + c
        pltpu.async_copy(e_hbm.at[0, w], src_v, sem)
        pltpu.async_copy(e_hbm.at[1, w], dst_v, sem)
        pltpu.async_copy(tab_hbm.at[pl.ds(s * rpt, rpt)],
                         tab_sh.at[pl.ds(s * rpt, rpt)], sem)
        _zero_fill(rows_v.at[0])
        for k in range(rpt // CH):
            pltpu.sync_copy(rows_v.at[0], acc_sh.at[pl.ds(s * rpt + k * CH, CH)])
        pltpu.make_async_copy(e_hbm.at[0, w], src_v, sem).wait()
        pltpu.make_async_copy(e_hbm.at[1, w], dst_v, sem).wait()
        pltpu.make_async_copy(tab_hbm.at[pl.ds(s * rpt, rpt)],
                              tab_sh.at[pl.ds(s * rpt, rpt)], sem).wait()
        plsc.subcore_barrier()

        for p in range(3):
            pltpu.async_copy(tab_sh.at[src_v.at[p]], rows_v.at[p], gsem.at[p])

        def step(j, _):
            b = lax.rem(j, 3)
            pltpu.make_async_copy(tab_sh.at[src_v.at[j]], rows_v.at[b],
                                  gsem.at[b]).wait()
            pltpu.async_copy(rows_v.at[b], acc_sh.at[dst_v.at[j]], ssem.at[b],
                             add=True)

            @pl.when(j + 3 < nch)
            def _():
                pltpu.make_async_copy(rows_v.at[b], acc_sh.at[dst_v.at[j]],
                                      ssem.at[b]).wait()
                pltpu.async_copy(tab_sh.at[src_v.at[j + 3]], rows_v.at[b],
                                 gsem.at[b])

            return 0

        lax.fori_loop(0, nch, step, 0)
        for p in range(3):
            j = nch - 3 + p
            b = j % 3
            pltpu.make_async_copy(rows_v.at[b], acc_sh.at[dst_v.at[j]],
                                  ssem.at[b]).wait()
        plsc.subcore_barrier()
        for k in range(rpt // CH):
            pltpu.async_copy(acc_sh.at[pl.ds(s * rpt + k * CH, CH)],
                             out_hbm.at[c, pl.ds(s * rpt + k * CH, CH)], sem)
        for k in range(rpt // CH):
            pltpu.make_async_copy(acc_sh.at[pl.ds(s * rpt + k * CH, CH)],
                                  out_hbm.at[c, pl.ds(s * rpt + k * CH, CH)],
                                  sem).wait()

    return pl.kernel(
        body,
        out_type=jax.ShapeDtypeStruct((NC, n_pad, h), F32),
        mesh=plsc.VectorSubcoreMesh(**_MESH),
        compiler_params=pltpu.CompilerParams(use_tc_tiling_on_sc=False),
        scratch_types=[
            pltpu.VMEM((nch, ch), I32),
            pltpu.VMEM((nch, ch), I32),
            pltpu.VMEM((3, CH, h), F32),
            pltpu.VMEM_SHARED((n_pad, h), F32),
            pltpu.VMEM_SHARED((n_pad, h), F32),
            pltpu.SemaphoreType.DMA,
            pltpu.SemaphoreType.DMA((3,)),
            pltpu.SemaphoreType.DMA((3,)),
        ],
    )(table, e3)


def _tc_k1(x, w1t, degp, n_pad):
    d = x.shape[1]
    h = w1t.shape[1]
    grid = (n_pad // RB,)

    def body(x_ref, w_ref, dg_ref, hp1s_ref, dinv_ref):
        dg = dg_ref[...]
        deg = dg[0, :, 0:1] + dg[1, :, 0:1] + 1.0
        dinv = lax.rsqrt(deg)
        hp1 = jnp.dot(x_ref[...], w_ref[...], preferred_element_type=F32)
        hp1s_ref[...] = hp1 * dinv
        dinv_ref[...] = dinv

    return pl.pallas_call(
        body,
        grid=grid,
        in_specs=[
            pl.BlockSpec((RB, d), lambda i: (i, 0)),
            pl.BlockSpec((d, h), lambda i: (0, 0)),
            pl.BlockSpec((NC, RB, LANES), lambda i: (0, i, 0)),
        ],
        out_specs=[
            pl.BlockSpec((RB, h), lambda i: (i, 0)),
            pl.BlockSpec((RB, 1), lambda i: (i, 0)),
        ],
        out_shape=[
            jax.ShapeDtypeStruct((n_pad, h), F32),
            jax.ShapeDtypeStruct((n_pad, 1), F32),
        ],
    )(x, w1t, degp)


def _tc_k2(s1, hp1s, dinv, b1, g1, be1, rm1, rv1, w2t):
    n_pad, h = hp1s.shape
    grid = (n_pad // RB,)

    def body(s_ref, hp_ref, dv_ref, b_ref, g_ref, be_ref, rm_ref, rv_ref,
             w_ref, out_ref):
        s = s_ref[...]
        dinv = dv_ref[...]
        conv = dinv * (s[0] + s[1] + hp_ref[...]) + b_ref[...]
        scale = g_ref[...] * lax.rsqrt(rv_ref[...] + 1e-5)
        hh = jnp.maximum((conv - rm_ref[...]) * scale + be_ref[...], 0.0)
        hp2 = jnp.dot(hh, w_ref[...], preferred_element_type=F32)
        out_ref[...] = hp2 * dinv

    vec = lambda: pl.BlockSpec((1, h), lambda i: (0, 0))
    return pl.pallas_call(
        body,
        grid=grid,
        in_specs=[
            pl.BlockSpec((NC, RB, h), lambda i: (0, i, 0)),
            pl.BlockSpec((RB, h), lambda i: (i, 0)),
            pl.BlockSpec((RB, 1), lambda i: (i, 0)),
            vec(), vec(), vec(), vec(), vec(),
            pl.BlockSpec((h, h), lambda i: (0, 0)),
        ],
        out_specs=pl.BlockSpec((RB, h), lambda i: (i, 0)),
        out_shape=jax.ShapeDtypeStruct((n_pad, h), F32),
    )(s1, hp1s, dinv, b1, g1, be1, rm1, rv1, w2t)


def _tc_k3(s2, hp2s, dinv, b2, g2, be2, rm2, rv2, fc1t, fc1b, fc2t, fc2b, n):
    n_pad, h = hp2s.shape
    h2 = fc1t.shape[1]
    grid = (n_pad // RB,)

    def body(s_ref, hp_ref, dv_ref, b_ref, g_ref, be_ref, rm_ref, rv_ref,
             f1_ref, f1b_ref, f2_ref, f2b_ref, out_ref):
        s = s_ref[...]
        dinv = dv_ref[...]
        conv = dinv * (s[0] + s[1] + hp_ref[...]) + b_ref[...]
        scale = g_ref[...] * lax.rsqrt(rv_ref[...] + 1e-5)
        hh = jnp.maximum((conv - rm_ref[...]) * scale + be_ref[...], 0.0)
        hh = jnp.maximum(
            jnp.dot(hh, f1_ref[...], preferred_element_type=F32) + f1b_ref[...],
            0.0)
        out_ref[...] = (jnp.dot(hh, f2_ref[...], preferred_element_type=F32)
                        + f2b_ref[...])

    vec = lambda k: pl.BlockSpec((1, k), lambda i: (0, 0))
    return pl.pallas_call(
        body,
        grid=grid,
        in_specs=[
            pl.BlockSpec((NC, RB, h), lambda i: (0, i, 0)),
            pl.BlockSpec((RB, h), lambda i: (i, 0)),
            pl.BlockSpec((RB, 1), lambda i: (i, 0)),
            vec(h), vec(h), vec(h), vec(h), vec(h),
            pl.BlockSpec((h, h2), lambda i: (0, 0)),
            vec(h2),
            pl.BlockSpec((h2, 1), lambda i: (0, 0)),
            vec(1),
        ],
        out_specs=pl.BlockSpec((RB, 1), lambda i: (i, 0)),
        out_shape=jax.ShapeDtypeStruct((n, 1), F32),
    )(s2, hp2s, dinv, b2, g2, be2, rm2, rv2, fc1t, fc1b, fc2t, fc2b)


def kernel(x, edge_index, W1, b1, W2, b2, g1, be1, rm1, rv1, g2, be2, rm2,
           rv2, fc1_w, fc1_b, fc2_w, fc2_b):
    n, d = x.shape
    e = edge_index.shape[1]

    n_pad = _pad_to(n + 1, NS * CH)
    e_pad = _pad_to(e, NW * CH)
    nch = e_pad // (NW * CH)

    e3 = jnp.pad(edge_index.astype(I32), ((0, 0), (0, e_pad - e)),
                 constant_values=n).reshape(2, NW, nch, CH)

    row = lambda v: v.reshape(1, -1)

    degp = _sc_degree(e3, n_pad)
    hp1s, dinv = _tc_k1(x, W1.T, degp, n_pad)
    s1 = _sc_scatter(hp1s, e3)
    hp2s = _tc_k2(s1, hp1s, dinv, row(b1), row(g1), row(be1), row(rm1),
                  row(rv1), W2.T)
    s2 = _sc_scatter(hp2s, e3)
    return _tc_k3(s2, hp2s, dinv, row(b2), row(g2), row(be2), row(rm2),
                  row(rv2), fc1_w.T, row(fc1_b), fc2_w.T, row(fc2_b), n)


# speedup vs baseline: 1.0328x; 1.0019x over previous
import functools

import jax
import jax.numpy as jnp
from jax import lax
from jax.experimental import pallas as pl
from jax.experimental.pallas import tpu as pltpu
from jax.experimental.pallas import tpu_sc as plsc

F32 = jnp.float32
I32 = jnp.int32

NC = 2
NS = 16
NW = NC * NS
LANES = 16
CH = 128
RB = 5120

_MESH = dict(core_axis_name="c", subcore_axis_name="s")


def _pad_to(v, m):
    return ((v + m - 1) // m) * m


def _zero_fill(ref):
    rows, cols = ref.shape
    z = jnp.zeros((LANES,), F32)

    def body(i, _):
        r = i // (cols // LANES)
        c = lax.rem(i, cols // LANES)
        ref[r, pl.ds(c * LANES, LANES)] = z
        return 0

    lax.fori_loop(0, rows * (cols // LANES), body, 0)


def _sc_degree(e3, n_pad):
    _, nw, nch, ch = e3.shape
    rpt = n_pad // NS

    def body(e_hbm, out_hbm, dst_v, ones_v, acc_sh):
        c = lax.axis_index("c")
        s = lax.axis_index("s")
        w = s * NC + c
        pltpu.sync_copy(e_hbm.at[1, w], dst_v)
        _zero_fill(ones_v)
        for k in range(rpt // CH):
            pltpu.sync_copy(ones_v, acc_sh.at[pl.ds(s * rpt + k * CH, CH)])
        one = jnp.ones((LANES,), F32)

        def fill(i, _):
            ones_v[i, pl.ds(0, LANES)] = one
            return 0

        lax.fori_loop(0, CH, fill, 0)
        plsc.subcore_barrier()

        def step(j, _):
            pltpu.sync_copy(ones_v, acc_sh.at[dst_v.at[j]], add=True)
            return 0

        lax.fori_loop(0, nch, step, 0)
        plsc.subcore_barrier()
        for k in range(rpt // CH):
            pltpu.sync_copy(acc_sh.at[pl.ds(s * rpt + k * CH, CH)],
                            out_hbm.at[c, pl.ds(s * rpt + k * CH, CH)])

    return pl.kernel(
        body,
        out_type=jax.ShapeDtypeStruct((NC, n_pad, LANES), F32),
        mesh=plsc.VectorSubcoreMesh(**_MESH),
        compiler_params=pltpu.CompilerParams(use_tc_tiling_on_sc=False),
        scratch_types=[
            pltpu.VMEM((nch, ch), I32),
            pltpu.VMEM((CH, LANES), F32),
            pltpu.VMEM_SHARED((n_pad, LANES), F32),
        ],
    )(e3)


def _sc_scatter(table, e3):
    n_pad, h = table.shape
    _, nw, nch, ch = e3.shape
    rpt = n_pad // NS

    def body(tab_hbm, e_hbm, out_hbm, src_v, dst_v, rows_v, acc_sh,
             tab_sh, sem, gsem, ssem):
        c = lax.axis_index("c")
        s = lax.axis_index("s")
        w = s * NC + c
        pltpu.async_copy(e_hbm.at[0, w], src_v, sem)
        pltpu.async_copy(e_hbm.at[1, w], dst_v, sem)
        pltpu.async_copy(tab_hbm.at[pl.ds(s * rpt, rpt)],
                         tab_sh.at[pl.ds(s * rpt, rpt)], sem)
        _zero_fill(rows_v.at[0])
        for k in range(rpt // CH):
            pltpu.sync_copy(rows_v.at[0], acc_sh.at[pl.ds(s * rpt + k * CH, CH)])
        pltpu.make_async_copy(e_hbm.at[0, w], src_v, sem).wait()
        pltpu.make_async_copy(e_hbm.at[1, w], dst_v, sem).wait()
        pltpu.make_async_copy(tab_hbm.at[pl.ds(s * rpt, rpt)],
                              tab_sh.at[pl.ds(s * rpt, rpt)], sem).wait()
        plsc.subcore_barrier()

        for p in range(3):
            pltpu.async_copy(tab_sh.at[src_v.at[p]], rows_v.at[p], gsem.at[p])

        def step(j, _):
            b = lax.rem(j, 3)
            pltpu.make_async_copy(tab_sh.at[src_v.at[j]], rows_v.at[b],
                                  gsem.at[b]).wait()
            pltpu.async_copy(rows_v.at[b], acc_sh.at[dst_v.at[j]], ssem.at[b],
                             add=True)

            @pl.when(j + 3 < nch)
            def _():
                pltpu.make_async_copy(rows_v.at[b], acc_sh.at[dst_v.at[j]],
                                      ssem.at[b]).wait()
                pltpu.async_copy(tab_sh.at[src_v.at[j + 3]], rows_v.at[b],
                                 gsem.at[b])

            return 0

        lax.fori_loop(0, nch, step, 0)
        for p in range(3):
            j = nch - 3 + p
            b = j % 3
            pltpu.make_async_copy(rows_v.at[b], acc_sh.at[dst_v.at[j]],
                                  ssem.at[b]).wait()
        plsc.subcore_barrier()
        for k in range(rpt // CH):
            pltpu.async_copy(acc_sh.at[pl.ds(s * rpt + k * CH, CH)],
                             out_hbm.at[c, pl.ds(s * rpt + k * CH, CH)], sem)
        for k in range(rpt // CH):
            pltpu.make_async_copy(acc_sh.at[pl.ds(s * rpt + k * CH, CH)],
                                  out_hbm.at[c, pl.ds(s * rpt + k * CH, CH)],
                                  sem).wait()

    return pl.kernel(
        body,
        out_type=jax.ShapeDtypeStruct((NC, n_pad, h), F32),
        mesh=plsc.VectorSubcoreMesh(**_MESH),
        compiler_params=pltpu.CompilerParams(use_tc_tiling_on_sc=False),
        scratch_types=[
            pltpu.VMEM((nch, ch), I32),
            pltpu.VMEM((nch, ch), I32),
            pltpu.VMEM((3, CH, h), F32),
            pltpu.VMEM_SHARED((n_pad, h), F32),
            pltpu.VMEM_SHARED((n_pad, h), F32),
            pltpu.SemaphoreType.DMA,
            pltpu.SemaphoreType.DMA((3,)),
            pltpu.SemaphoreType.DMA((3,)),
        ],
    )(table, e3)


def _tc_k1(x, w1t, degp, n_pad):
    d = x.shape[1]
    h = w1t.shape[1]
    grid = (n_pad // RB,)

    def body(x_ref, w_ref, dg_ref, hp1s_ref, dinv_ref):
        dg = dg_ref[...]
        deg = dg[0, :, 0:1] + dg[1, :, 0:1] + 1.0
        dinv = lax.rsqrt(deg)
        hp1 = jnp.dot(x_ref[...], w_ref[...], preferred_element_type=F32)
        hp1s_ref[...] = hp1 * dinv
        dinv_ref[...] = dinv

    return pl.pallas_call(
        body,
        grid=grid,
        in_specs=[
            pl.BlockSpec((RB, d), lambda i: (i, 0)),
            pl.BlockSpec((d, h), lambda i: (0, 0)),
            pl.BlockSpec((NC, RB, LANES), lambda i: (0, i, 0)),
        ],
        out_specs=[
            pl.BlockSpec((RB, h), lambda i: (i, 0)),
            pl.BlockSpec((RB, 1), lambda i: (i, 0)),
        ],
        out_shape=[
            jax.ShapeDtypeStruct((n_pad, h), F32),
            jax.ShapeDtypeStruct((n_pad, 1), F32),
        ],
    )(x, w1t, degp)


def _tc_k2(s1, hp1s, dinv, b1, g1, be1, rm1, rv1, w2t):
    n_pad, h = hp1s.shape
    grid = (n_pad // RB,)

    def body(s_ref, hp_ref, dv_ref, b_ref, g_ref, be_ref, rm_ref, rv_ref,
             w_ref, out_ref):
        s = s_ref[...]
        dinv = dv_ref[...]
        conv = dinv * (s[0] + s[1] + hp_ref[...]) + b_ref[...]
        scale = g_ref[...] * lax.rsqrt(rv_ref[...] + 1e-5)
        hh = jnp.maximum((conv - rm_ref[...]) * scale + be_ref[...], 0.0)
        hp2 = jnp.dot(hh, w_ref[...], preferred_element_type=F32)
        out_ref[...] = hp2 * dinv

    vec = lambda: pl.BlockSpec((1, h), lambda i: (0, 0))
    return pl.pallas_call(
        body,
        grid=grid,
        in_specs=[
            pl.BlockSpec((NC, RB, h), lambda i: (0, i, 0)),
            pl.BlockSpec((RB, h), lambda i: (i, 0)),
            pl.BlockSpec((RB, 1), lambda i: (i, 0)),
            vec(), vec(), vec(), vec(), vec(),
            pl.BlockSpec((h, h), lambda i: (0, 0)),
        ],
        out_specs=pl.BlockSpec((RB, h), lambda i: (i, 0)),
        out_shape=jax.ShapeDtypeStruct((n_pad, h), F32),
    )(s1, hp1s, dinv, b1, g1, be1, rm1, rv1, w2t)


def _tc_k3(s2, hp2s, dinv, b2, g2, be2, rm2, rv2, fc1t, fc1b, fc2t, fc2b, n):
    n_pad, h = hp2s.shape
    h2 = fc1t.shape[1]
    grid = (n_pad // RB,)

    def body(s_ref, hp_ref, dv_ref, b_ref, g_ref, be_ref, rm_ref, rv_ref,
             f1_ref, f1b_ref, f2_ref, f2b_ref, out_ref):
        s = s_ref[...]
        dinv = dv_ref[...]
        conv = dinv * (s[0] + s[1] + hp_ref[...]) + b_ref[...]
        scale = g_ref[...] * lax.rsqrt(rv_ref[...] + 1e-5)
        hh = jnp.maximum((conv - rm_ref[...]) * scale + be_ref[...], 0.0)
        hh = jnp.maximum(
            jnp.dot(hh, f1_ref[...], preferred_element_type=F32) + f1b_ref[...],
            0.0)
        out_ref[...] = (jnp.dot(hh, f2_ref[...], preferred_element_type=F32)
                        + f2b_ref[...])

    vec = lambda k: pl.BlockSpec((1, k), lambda i: (0, 0))
    return pl.pallas_call(
        body,
        grid=grid,
        in_specs=[
            pl.BlockSpec((NC, RB, h), lambda i: (0, i, 0)),
            pl.BlockSpec((RB, h), lambda i: (i, 0)),
            pl.BlockSpec((RB, 1), lambda i: (i, 0)),
            vec(h), vec(h), vec(h), vec(h), vec(h),
            pl.BlockSpec((h, h2), lambda i: (0, 0)),
            vec(h2),
            pl.BlockSpec((h2, 1), lambda i: (0, 0)),
            vec(1),
        ],
        out_specs=pl.BlockSpec((RB, 1), lambda i: (i, 0)),
        out_shape=jax.ShapeDtypeStruct((n, 1), F32),
    )(s2, hp2s, dinv, b2, g2, be2, rm2, rv2, fc1t, fc1b, fc2t, fc2b)


def kernel(x, edge_index, W1, b1, W2, b2, g1, be1, rm1, rv1, g2, be2, rm2,
           rv2, fc1_w, fc1_b, fc2_w, fc2_b):
    n, d = x.shape
    e = edge_index.shape[1]

    n_pad = _pad_to(n + 1, NS * CH)
    e_pad = _pad_to(e, NW * CH)
    nch = e_pad // (NW * CH)

    e3 = jnp.pad(edge_index.astype(I32), ((0, 0), (0, e_pad - e)),
                 constant_values=n).reshape(2, NW, nch, CH)

    row = lambda v: v.reshape(1, -1)

    degp = _sc_degree(e3, n_pad)
    hp1s, dinv = _tc_k1(x, W1.T, degp, n_pad)
    s1 = _sc_scatter(hp1s, e3)
    hp2s = _tc_k2(s1, hp1s, dinv, row(b1), row(g1), row(be1), row(rm1),
                  row(rv1), W2.T)
    s2 = _sc_scatter(hp2s, e3)
    return _tc_k3(s2, hp2s, dinv, row(b2), row(g2), row(be2), row(rm2),
                  row(rv2), fc1_w.T, row(fc1_b), fc2_w.T, row(fc2_b), n)
